# Initial kernel scaffold; baseline (speedup 1.0000x reference)
import jax
import jax.numpy as jnp
from jax.experimental import pallas as pl


def kernel(kp1, w_kp1, kp1_desc, desc2):
    raise NotImplementedError("write your pallas kernel here")


# speedup vs baseline: 16.7535x; 16.7535x over previous
import functools

import jax
import jax.numpy as jnp
from jax.experimental import pallas as pl
from jax.experimental.pallas import tpu as pltpu

GRID_SIZE = 16
MARGIN = 1.0
LOSS_LAMBDA = 1.0

BN = 256


def _body(w_ref, kd_ref, d2t_ref, d2_ref, loss_ref, pos_ref, *, n_rows, m, h, w, cp):
    i = pl.program_id(0)
    wblk = w_ref[0]
    py = wblk[:, 0:1]
    px = wblk[:, 1:2]
    kd = kd_ref[...]
    d2t = d2t_ref[...]
    d2 = d2_ref[...]

    ys = py / GRID_SIZE - 0.5
    xs = px / GRID_SIZE - 0.5
    x0 = jnp.floor(xs)
    y0 = jnp.floor(ys)
    x1 = x0 + 1.0
    y1 = y0 + 1.0
    wx1 = xs - x0
    wx0 = 1.0 - wx1
    wy1 = ys - y0
    wy0 = 1.0 - wy1

    lane = jax.lax.broadcasted_iota(jnp.int32, (n_rows, m), 1)

    def wmat(yf, xf, wgt):
        valid = (yf >= 0) & (yf <= h - 1) & (xf >= 0) & (xf <= w - 1)
        yc = jnp.clip(yf, 0, h - 1).astype(jnp.int32)
        xc = jnp.clip(xf, 0, w - 1).astype(jnp.int32)
        idx = yc * w + xc
        return jnp.where(lane == idx, wgt * valid.astype(jnp.float32), 0.0)

    wm = (wmat(y0, x0, wy0 * wx0) + wmat(y0, x1, wy0 * wx1)
          + wmat(y1, x0, wy1 * wx0) + wmat(y1, x1, wy1 * wx1))

    samp = jax.lax.dot_general(wm, d2, (((1,), (0,)), ((), ())),
                               preferred_element_type=jnp.float32,
                               precision=jax.lax.Precision.HIGHEST)
    nrm = jnp.sqrt(jnp.sum(samp * samp, axis=1, keepdims=True))
    samp = samp / jnp.maximum(nrm, 1e-12)
    pos = 2.0 - 2.0 * jnp.sum(kd * samp, axis=1, keepdims=True)

    sim = 2.0 - 2.0 * jax.lax.dot_general(kd, d2t, (((1,), (0,)), ((), ())),
                                          preferred_element_type=jnp.float32,
                                          precision=jax.lax.Precision.HIGHEST)

    cx = (lane % w).astype(jnp.float32) * GRID_SIZE + GRID_SIZE // 2
    cy = (lane // w).astype(jnp.float32) * GRID_SIZE + GRID_SIZE // 2
    dist = jnp.sqrt((px - cx) ** 2 + (py - cy) ** 2)
    work = dist
    for _ in range(4):
        mn = jnp.min(work, axis=1, keepdims=True)
        sel = jnp.min(jnp.where(work == mn, lane, m), axis=1, keepdims=True)
        hit = lane == sel
        sim = sim + jnp.where(hit, 5.0, 0.0)
        work = jnp.where(hit, jnp.inf, work)

    lsum = jnp.zeros((), jnp.float32)
    work = sim
    for k in range(4):
        mn = jnp.min(work, axis=1, keepdims=True)
        lsum += jnp.sum(jnp.maximum(pos - mn + MARGIN, 0.0))
        if k < 3:
            sel = jnp.min(jnp.where(work == mn, lane, m), axis=1, keepdims=True)
            work = jnp.where(lane == sel, jnp.inf, work)

    psum = jnp.sum(pos)

    @pl.when(i == 0)
    def _():
        loss_ref[...] = jnp.zeros((1, 1), jnp.float32)
        pos_ref[...] = jnp.zeros((1, 1), jnp.float32)

    n_total = pl.num_programs(0) * n_rows
    loss_ref[...] += jnp.full((1, 1), lsum * (LOSS_LAMBDA / (4.0 * n_total)))
    pos_ref[...] += jnp.full((1, 1), psum / n_total)


@jax.jit
def kernel(kp1, w_kp1, kp1_desc, desc2):
    del kp1
    n, c = kp1_desc.shape
    _, _, hc, wc = desc2.shape
    m = hc * wc
    cp = 256
    g = n // BN

    d2 = jnp.transpose(desc2[0], (1, 2, 0)).reshape(m, c)
    d2p = jnp.pad(d2, ((0, 0), (0, cp - c)))
    kdp = jnp.pad(kp1_desc, ((0, 0), (0, cp - c)))
    d2t = d2p.T
    w3 = w_kp1.reshape(g, BN, 2)

    body = functools.partial(_body, n_rows=BN, m=m, h=hc, w=wc, cp=cp)
    loss, posm = pl.pallas_call(
        body,
        grid=(g,),
        in_specs=[
            pl.BlockSpec((1, BN, 2), lambda i: (i, 0, 0)),
            pl.BlockSpec((BN, cp), lambda i: (i, 0)),
            pl.BlockSpec((cp, m), lambda i: (0, 0)),
            pl.BlockSpec((m, cp), lambda i: (0, 0)),
        ],
        out_specs=[
            pl.BlockSpec((1, 1), lambda i: (0, 0)),
            pl.BlockSpec((1, 1), lambda i: (0, 0)),
        ],
        out_shape=[
            jax.ShapeDtypeStruct((1, 1), jnp.float32),
            jax.ShapeDtypeStruct((1, 1), jnp.float32),
        ],
    )(w3, kdp, d2t, d2p)
    return (loss[0, 0], posm[0, 0])
